# baseline (speedup 1.0000x reference)
import functools

import jax
import jax.numpy as jnp
from jax.experimental import pallas as pl
from jax.experimental.pallas import tpu as pltpu
from jax.experimental.pallas import tpu_sc as plsc

_SMOOTH = 0.1
_PAD = 0
_CONF = 1.0 - _SMOOTH


def _rowstats_kernel(x_ref, y_ref, p_ref, lse_ref, m_sc, s_sc, t_sc, x0_sc,
                     *, C, Cb, n_cb):
    j = pl.program_id(1)
    xb = x_ref[...]

    @pl.when(j == 0)
    def _init():
        bmax = jnp.max(xb, axis=1, keepdims=True)
        m_sc[...] = bmax
        s_sc[...] = jnp.sum(jnp.exp(xb - bmax), axis=1, keepdims=True)
        t_sc[...] = jnp.sum(xb, axis=1, keepdims=True)
        x0_sc[...] = xb[:, 0:1]

    @pl.when(jnp.logical_and(j > 0, j < n_cb - 1))
    def _acc():
        bmax = jnp.max(xb, axis=1, keepdims=True)
        m_old = m_sc[...]
        m_new = jnp.maximum(m_old, bmax)
        s_sc[...] = (s_sc[...] * jnp.exp(m_old - m_new)
                     + jnp.sum(jnp.exp(xb - m_new), axis=1, keepdims=True))
        m_sc[...] = m_new
        t_sc[...] = t_sc[...] + jnp.sum(xb, axis=1, keepdims=True)

    @pl.when(j == n_cb - 1)
    def _tail_and_emit():
        cols = jax.lax.broadcasted_iota(jnp.int32, xb.shape, 1) + j * Cb
        valid = cols < C
        xv = jnp.where(valid, xb, -jnp.inf)
        bmax = jnp.max(xv, axis=1, keepdims=True)
        m_old = m_sc[...]
        m_new = jnp.maximum(m_old, bmax)
        s = (s_sc[...] * jnp.exp(m_old - m_new)
             + jnp.sum(jnp.exp(xv - m_new), axis=1, keepdims=True))
        t = t_sc[...] + jnp.sum(jnp.where(valid, xb, 0.0), axis=1,
                                keepdims=True)

        eps = _SMOOTH / (C - 2)
        K = _SMOOTH * jnp.log(eps) + _CONF * jnp.log(_CONF)
        lse = m_new + jnp.log(s)
        ssum = t - C * lse
        logp0 = x0_sc[...] - lse
        live = y_ref[...] != _PAD
        p_ref[...] = jnp.where(live, K - eps * (ssum - logp0), 0.0)
        lse_ref[...] = jnp.where(live, lse, 0.0)


def _sc_gather_kernel(xflat_hbm, idx_hbm, o_hbm, *, window, n_win):
    def body(i_vmem, o_vmem):
        pltpu.sync_copy(xflat_hbm.at[i_vmem.at[0]], o_vmem)

    pltpu.emit_pipeline(
        body,
        grid=(n_win,),
        in_specs=[pl.BlockSpec((1, window), index_map=lambda i: (0, i))],
        out_specs=[pl.BlockSpec((window, 128), index_map=lambda i: (i, 0))],
        core_axis_name=("core", "subcore"),
        dimension_semantics=(pltpu.PARALLEL,),
    )(idx_hbm, o_hbm)


@jax.jit
def kernel(x, y):
    B, C = x.shape
    Rb, Cb = 512, 2048
    n_rb = B // Rb
    n_cb = pl.cdiv(C, Cb)
    y2 = y.astype(jnp.int32).reshape(B, 1)

    window = 128
    n_win = B // window
    lanes = 128
    flat_idx = (jnp.arange(B, dtype=jnp.int32) * C + y.astype(jnp.int32))
    row_idx = (flat_idx // lanes).reshape(1, B)
    xflat = x.reshape(B * C // lanes, lanes)
    sc_mesh = plsc.VectorSubcoreMesh(core_axis_name="core",
                                     subcore_axis_name="subcore")
    gath_rows = pl.kernel(
        functools.partial(_sc_gather_kernel, window=window, n_win=n_win),
        out_type=jax.ShapeDtypeStruct((B, lanes), x.dtype),
        mesh=sc_mesh,
    )(xflat, row_idx)
    gath = jnp.take_along_axis(gath_rows, (flat_idx % lanes)[:, None],
                               axis=1)

    partial_loss, lse = pl.pallas_call(
        functools.partial(_rowstats_kernel, C=C, Cb=Cb, n_cb=n_cb),
        grid=(n_rb, n_cb),
        in_specs=[
            pl.BlockSpec((Rb, Cb), lambda i, j: (i, j)),
            pl.BlockSpec((Rb, 1), lambda i, j: (i, 0)),
        ],
        out_specs=[
            pl.BlockSpec((Rb, 1), lambda i, j: (i, 0)),
            pl.BlockSpec((Rb, 1), lambda i, j: (i, 0)),
        ],
        out_shape=[
            jax.ShapeDtypeStruct((B, 1), x.dtype),
            jax.ShapeDtypeStruct((B, 1), x.dtype),
        ],
        scratch_shapes=[pltpu.VMEM((Rb, 1), jnp.float32)] * 4,
        compiler_params=pltpu.CompilerParams(
            dimension_semantics=("parallel", "arbitrary"),
        ),
    )(x, y2)

    eps = _SMOOTH / (C - 2)
    live = (y2 != _PAD)
    logpy = jnp.where(live, gath - lse, 0.0)
    return jnp.sum(partial_loss) + (eps - _CONF) * jnp.sum(logpy)


# speedup vs baseline: 2.0517x; 2.0517x over previous
import functools

import jax
import jax.numpy as jnp
from jax.experimental import pallas as pl
from jax.experimental.pallas import tpu as pltpu

_SMOOTH = 0.1
_PAD = 0
_CONF = 1.0 - _SMOOTH


def _rowstats_kernel(x_ref, y_ref, out_ref, m_sc, s_sc, t_sc, g_sc, x0_sc,
                     *, C, Cb, n_cb):
    j = pl.program_id(1)
    xb = x_ref[...]
    yb = y_ref[...]
    cols_local = jax.lax.broadcasted_iota(jnp.int32, xb.shape, 1)
    hit = cols_local == yb - j * Cb
    bg = jnp.sum(jnp.where(hit, xb, 0.0), axis=1, keepdims=True)

    @pl.when(j == 0)
    def _init():
        bmax = jnp.max(xb, axis=1, keepdims=True)
        m_sc[...] = bmax
        s_sc[...] = jnp.sum(jnp.exp(xb - bmax), axis=1, keepdims=True)
        t_sc[...] = jnp.sum(xb, axis=1, keepdims=True)
        g_sc[...] = bg
        x0_sc[...] = xb[:, 0:1]

    @pl.when(jnp.logical_and(j > 0, j < n_cb - 1))
    def _acc():
        bmax = jnp.max(xb, axis=1, keepdims=True)
        m_old = m_sc[...]
        m_new = jnp.maximum(m_old, bmax)
        s_sc[...] = (s_sc[...] * jnp.exp(m_old - m_new)
                     + jnp.sum(jnp.exp(xb - m_new), axis=1, keepdims=True))
        m_sc[...] = m_new
        t_sc[...] = t_sc[...] + jnp.sum(xb, axis=1, keepdims=True)
        g_sc[...] = g_sc[...] + bg

    @pl.when(j == n_cb - 1)
    def _tail_and_emit():
        valid = cols_local < C - j * Cb
        xv = jnp.where(valid, xb, -jnp.inf)
        bmax = jnp.max(xv, axis=1, keepdims=True)
        m_old = m_sc[...]
        m_new = jnp.maximum(m_old, bmax)
        s = (s_sc[...] * jnp.exp(m_old - m_new)
             + jnp.sum(jnp.exp(xv - m_new), axis=1, keepdims=True))
        t = t_sc[...] + jnp.sum(jnp.where(valid, xb, 0.0), axis=1,
                                keepdims=True)
        g = g_sc[...] + bg

        eps = _SMOOTH / (C - 2)
        K = _SMOOTH * jnp.log(eps) + _CONF * jnp.log(_CONF)
        lse = m_new + jnp.log(s)
        ssum = t - C * lse
        logp0 = x0_sc[...] - lse
        logpy = g - lse
        row = K - eps * (ssum - logp0 - logpy) - _CONF * logpy
        out_ref[...] = jnp.where(yb != _PAD, row, 0.0)


@jax.jit
def kernel(x, y):
    B, C = x.shape
    Rb, Cb = 512, 2048
    n_rb = B // Rb
    n_cb = pl.cdiv(C, Cb)
    y2 = y.astype(jnp.int32).reshape(B, 1)

    rows = pl.pallas_call(
        functools.partial(_rowstats_kernel, C=C, Cb=Cb, n_cb=n_cb),
        grid=(n_rb, n_cb),
        in_specs=[
            pl.BlockSpec((Rb, Cb), lambda i, j: (i, j)),
            pl.BlockSpec((Rb, 1), lambda i, j: (i, 0)),
        ],
        out_specs=pl.BlockSpec((Rb, 1), lambda i, j: (i, 0)),
        out_shape=jax.ShapeDtypeStruct((B, 1), x.dtype),
        scratch_shapes=[pltpu.VMEM((Rb, 1), jnp.float32)] * 5,
        compiler_params=pltpu.CompilerParams(
            dimension_semantics=("parallel", "arbitrary"),
        ),
    )(x, y2)
    return jnp.sum(rows)
